# baseline (speedup 1.0000x reference)
import functools

import jax
import jax.numpy as jnp
from jax import lax
from jax.experimental import pallas as pl
from jax.experimental.pallas import tpu as pltpu
from jax.experimental.pallas import tpu_sc as plsc

E = 6_400_000
G = 4096
NC = 2
NS = 16
NW = NC * NS
EPW = E // NW
NBUF = 5
CHUNK = 4_000
NCHUNK = EPW // CHUNK
L = 16
VPC = CHUNK // L

_F32 = jnp.float32
_I32 = jnp.int32


def _sc_partials(distance, eps_ct, lam_ct, r_star, edge_batch):
    mesh = plsc.VectorSubcoreMesh(core_axis_name="c", subcore_axis_name="s")

    buf_types = [
        pltpu.VMEM((CHUNK,), dt)
        for _ in range(NBUF)
        for dt in (_F32, _F32, _F32, _F32, _I32)
    ]

    @functools.partial(
        pl.kernel,
        mesh=mesh,
        out_type=jax.ShapeDtypeStruct((NW, G), _F32),
        compiler_params=pltpu.CompilerParams(needs_layout_passes=False),
        scratch_types=buf_types + [pltpu.VMEM((G,), _F32)]
        + [pltpu.SemaphoreType.DMA] * NBUF,
    )
    def k(*refs):
        hbms = refs[0:5]
        out_hbm = refs[5]
        bufs = tuple(refs[6 + 5 * i:11 + 5 * i] for i in range(NBUF))
        acc = refs[6 + 5 * NBUF]
        sems = refs[7 + 5 * NBUF:7 + 5 * NBUF + NBUF]
        cid = lax.axis_index("c")
        sid = lax.axis_index("s")
        wid = cid * NS + sid

        zero = jnp.zeros((L,), _F32)

        @plsc.parallel_loop(0, G // L, 1, unroll=8)
        def _(i):
            acc[pl.ds(i * L, L)] = zero

        base = wid * EPW
        lane = lax.iota(_I32, L)
        shift = jnp.minimum(lane + 1, L - 1)
        is_last = lane == (L - 1)

        def issue(ci, slot):
            start = base + ci * CHUNK
            for hbm, buf in zip(hbms, bufs[slot]):
                pltpu.async_copy(hbm.at[pl.ds(start, CHUNK)], buf, sems[slot])

        def drain(slot):
            src = pl.ds(0, CHUNK)
            for hbm, buf in zip(hbms, bufs[slot]):
                pltpu.make_async_copy(hbm.at[src], buf, sems[slot]).wait()

        def do_vec(slot, o):
            d_v, ep_v, lm_v, rs_v, b_v = bufs[slot]
            d = d_v[pl.ds(o, L)]
            ep = ep_v[pl.ds(o, L)]
            lm = lm_v[pl.ds(o, L)]
            rs = rs_v[pl.ds(o, L)]
            ib = b_v[pl.ds(o, L)]
            r = jnp.maximum(d, 1e-6)
            t = 1.0 / r
            ratio = lm * jnp.maximum(rs, 1e-6) * t
            t2 = t * t
            r3 = ratio * ratio * ratio
            pe = (0.5 * ep) * (t2 * t2) * jnp.exp(-r3)
            s = plsc.cumsum(pe)
            ib_next = jnp.take_along_axis(ib, shift, axis=0)
            is_bnd = ib != ib_next
            plsc.addupdate_scatter(acc, [ib], s, mask=is_bnd | is_last)
            plsc.addupdate_scatter(acc, [ib_next], -s, mask=is_bnd)

        def compute(slot):
            @plsc.parallel_loop(0, VPC, 1, unroll=8)
            def _(vi):
                do_vec(slot, vi * L)

        for p in range(NBUF - 1):
            issue(p, p)

        def ring_body(ti, carry):
            ci0 = ti * NBUF
            for p in range(NBUF):
                ci = ci0 + p
                drain(p)
                compute(p)
                nxt = ci + (NBUF - 1)

                @pl.when(nxt < NCHUNK)
                def _(p=p, nxt=nxt):
                    issue(nxt, (p + NBUF - 1) % NBUF)

            return carry

        lax.fori_loop(0, NCHUNK // NBUF, ring_body, 0)
        pltpu.sync_copy(acc, out_hbm.at[wid])

    return k(distance, eps_ct, lam_ct, r_star, edge_batch)


def _tc_reduce(partials):
    def body(p_ref, o_ref):
        o_ref[...] = jnp.sum(p_ref[...], axis=0, keepdims=True)

    out = pl.pallas_call(
        body,
        out_shape=jax.ShapeDtypeStruct((1, G), _F32),
    )(partials)
    return out.reshape(G)


def kernel(distance, eps_ct_ij, lam_ct_ij, r_star_ij, edge_batch, num_graphs):
    del num_graphs
    partials = _sc_partials(distance, eps_ct_ij, lam_ct_ij, r_star_ij,
                            edge_batch)
    return _tc_reduce(partials)


# speedup vs baseline: 1.0571x; 1.0571x over previous
import functools

import jax
import jax.numpy as jnp
from jax import lax
from jax.experimental import pallas as pl
from jax.experimental.pallas import tpu as pltpu
from jax.experimental.pallas import tpu_sc as plsc

E = 6_400_000
G = 4096
NC = 2
NS = 16
NW = NC * NS
EPW = E // NW
NBUF = 4
CHUNK = 5_000
NCHUNK = EPW // CHUNK
L = 16
VPC = CHUNK // L

_F32 = jnp.float32
_I32 = jnp.int32


def _sc_partials(distance, eps_ct, lam_ct, r_star, edge_batch):
    mesh = plsc.VectorSubcoreMesh(core_axis_name="c", subcore_axis_name="s")

    buf_types = [
        pltpu.VMEM((CHUNK,), dt)
        for _ in range(NBUF)
        for dt in (_F32, _F32, _F32, _F32, _I32)
    ]

    @functools.partial(
        pl.kernel,
        mesh=mesh,
        out_type=jax.ShapeDtypeStruct((NW, G), _F32),
        compiler_params=pltpu.CompilerParams(needs_layout_passes=False),
        scratch_types=buf_types + [pltpu.VMEM((G,), _F32)]
        + [pltpu.SemaphoreType.DMA] * NBUF,
    )
    def k(*refs):
        hbms = refs[0:5]
        out_hbm = refs[5]
        bufs = tuple(refs[6 + 5 * i:11 + 5 * i] for i in range(NBUF))
        acc = refs[6 + 5 * NBUF]
        sems = refs[7 + 5 * NBUF:7 + 5 * NBUF + NBUF]
        cid = lax.axis_index("c")
        sid = lax.axis_index("s")
        wid = cid * NS + sid

        zero = jnp.zeros((L,), _F32)

        @plsc.parallel_loop(0, G // L, 1, unroll=8)
        def _(i):
            acc[pl.ds(i * L, L)] = zero

        base = wid * EPW
        lane = lax.iota(_I32, L)
        shift = jnp.minimum(lane + 1, L - 1)
        is_last = lane == (L - 1)

        def issue(ci, slot):
            start = base + ci * CHUNK
            for hbm, buf in zip(hbms, bufs[slot]):
                pltpu.async_copy(hbm.at[pl.ds(start, CHUNK)], buf, sems[slot])

        def drain(slot):
            src = pl.ds(0, CHUNK)
            for hbm, buf in zip(hbms, bufs[slot]):
                pltpu.make_async_copy(hbm.at[src], buf, sems[slot]).wait()

        def do_vec(slot, o):
            d_v, ep_v, lm_v, rs_v, b_v = bufs[slot]
            d = d_v[pl.ds(o, L)]
            ep = ep_v[pl.ds(o, L)]
            lm = lm_v[pl.ds(o, L)]
            rs = rs_v[pl.ds(o, L)]
            ib = b_v[pl.ds(o, L)]
            r = jnp.maximum(d, 1e-6)
            t = 1.0 / r
            ratio = lm * jnp.maximum(rs, 1e-6) * t
            t2 = t * t
            r3 = ratio * ratio * ratio
            pe = (0.5 * ep) * (t2 * t2) * jnp.exp(-r3)
            s = plsc.cumsum(pe)
            ib_next = jnp.take_along_axis(ib, shift, axis=0)
            is_bnd = ib != ib_next
            plsc.addupdate_scatter(acc, [ib], s, mask=is_bnd | is_last)
            plsc.addupdate_scatter(acc, [ib_next], -s, mask=is_bnd)

        def compute(slot):
            @plsc.parallel_loop(0, VPC, 1, unroll=8)
            def _(vi):
                do_vec(slot, vi * L)

        for p in range(NBUF - 1):
            issue(p, p)

        def ring_body(ti, carry):
            ci0 = ti * NBUF
            for p in range(NBUF):
                ci = ci0 + p
                drain(p)
                nxt = ci + (NBUF - 1)

                @pl.when(nxt < NCHUNK)
                def _(p=p, nxt=nxt):
                    issue(nxt, (p + NBUF - 1) % NBUF)

                compute(p)

            return carry

        lax.fori_loop(0, NCHUNK // NBUF, ring_body, 0)
        pltpu.sync_copy(acc, out_hbm.at[wid])

    return k(distance, eps_ct, lam_ct, r_star, edge_batch)


def _tc_reduce(partials):
    def body(p_ref, o_ref):
        o_ref[...] = jnp.sum(p_ref[...], axis=0, keepdims=True)

    out = pl.pallas_call(
        body,
        out_shape=jax.ShapeDtypeStruct((1, G), _F32),
    )(partials)
    return out.reshape(G)


def kernel(distance, eps_ct_ij, lam_ct_ij, r_star_ij, edge_batch, num_graphs):
    del num_graphs
    partials = _sc_partials(distance, eps_ct_ij, lam_ct_ij, r_star_ij,
                            edge_batch)
    return _tc_reduce(partials)
